# baseline (speedup 1.0000x reference)
import jax
import jax.numpy as jnp
from jax import lax
from jax.experimental import pallas as pl
from jax.experimental.pallas import tpu as pltpu
from jax.experimental.pallas import tpu_sc as plsc

_N_NODES = 10000
_N_EDGES = 320000
_D = 128
_NCLS = 64
_B = 10000

_NC = 2
_NS = 16
_NW = _NC * _NS

_PEDGES = _N_EDGES // _NW
_CHUNK = 80
_NCH = _PEDGES // _CHUNK
_NBUF = 3

_EXP = 624
_EXP_TAIL = _N_NODES - _NS * _EXP

_BPAD = 10240
_PERW = _BPAD // _NW
_GCH = 80
_NGCH = _PERW // _GCH

_mesh = plsc.VectorSubcoreMesh(
    core_axis_name="c", subcore_axis_name="s", num_cores=_NC, num_subcores=_NS)


def _scatter_body(esrc_ref, edst_ref, x_ref, zx_ref, zdeg_ref,
                  psum_ref, pdeg_ref, sall, dall, acc):
    c = lax.axis_index("c")
    s = lax.axis_index("s")
    wid = s * _NC + c

    row0 = s * _EXP
    pltpu.sync_copy(zx_ref, acc.at[pl.ds(row0, _EXP)])

    @pl.when(s == _NS - 1)
    def _init_tail():
        pltpu.sync_copy(zx_ref.at[pl.ds(0, _EXP_TAIL)],
                        acc.at[pl.ds(_NS * _EXP, _EXP_TAIL)])

    e0 = wid * _PEDGES
    pltpu.sync_copy(esrc_ref.at[pl.ds(e0, _PEDGES)], sall)
    pltpu.sync_copy(edst_ref.at[pl.ds(e0, _PEDGES)], dall)
    plsc.subcore_barrier()

    def _main(rows, g0, g1, g2, s0, s1, s2):
        gsem = [g0, g1, g2][:_NBUF]
        ssem = [s0, s1, s2][:_NBUF]

        def _issue(j, b):
            pltpu.async_copy(x_ref.at[sall.at[pl.ds(j * _CHUNK, _CHUNK)]],
                             rows.at[b], gsem[b])

        def _consume(j, b):
            pltpu.make_async_copy(x_ref.at[pl.ds(0, _CHUNK)],
                                  rows.at[b], gsem[b]).wait()
            pltpu.async_copy(rows.at[b],
                             acc.at[dall.at[pl.ds(j * _CHUNK, _CHUNK)]],
                             ssem[b], add=True)

        def _drain_scatter(b):
            pltpu.make_async_copy(rows.at[b], acc.at[pl.ds(0, _CHUNK)],
                                  ssem[b]).wait()

        for b in range(_NBUF):
            _issue(b, b)

        def group_step(g, carry):
            for b in range(_NBUF):
                j = g * _NBUF + b
                _consume(j, b)
                nxt = j + _NBUF

                @pl.when(nxt < _NCH)
                def _reissue():
                    _drain_scatter(b)
                    _issue(nxt, b)
            return carry
        lax.fori_loop(0, _NCH // _NBUF, group_step, 0)
        for t, j in enumerate(range((_NCH // _NBUF) * _NBUF, _NCH)):
            _consume(j, t)
        for b in range(_NBUF):
            _drain_scatter(b)

    pl.run_scoped(
        _main,
        pltpu.VMEM((_NBUF, _CHUNK, _D), jnp.float32),
        *([pltpu.SemaphoreType.DMA] * 6),
    )

    def _deg(degloc):
        pltpu.sync_copy(zdeg_ref, degloc)
        one16 = jnp.ones((16,), jnp.float32)

        def step(i, carry):
            dvec = dall[pl.ds(i * 16, 16)]
            plsc.addupdate_scatter(degloc, [dvec], one16)
            return carry
        lax.fori_loop(0, _PEDGES // 16, step, 0)
        pltpu.sync_copy(degloc, pdeg_ref.at[wid])

    pl.run_scoped(_deg, pltpu.VMEM((_N_NODES,), jnp.float32))

    plsc.subcore_barrier()
    pltpu.sync_copy(acc.at[pl.ds(row0, _EXP)],
                    psum_ref.at[c, pl.ds(row0, _EXP)])

    @pl.when(s == _NS - 1)
    def _exp_tail():
        pltpu.sync_copy(acc.at[pl.ds(_NS * _EXP, _EXP_TAIL)],
                        psum_ref.at[c, pl.ds(_NS * _EXP, _EXP_TAIL)])


_scatter_call = pl.kernel(
    _scatter_body,
    out_type=(
        jax.ShapeDtypeStruct((_NC, _N_NODES, _D), jnp.float32),
        jax.ShapeDtypeStruct((_NW, _N_NODES), jnp.float32),
    ),
    mesh=_mesh,
    compiler_params=pltpu.CompilerParams(needs_layout_passes=False),
    scratch_types=(
        pltpu.VMEM((_PEDGES,), jnp.int32),
        pltpu.VMEM((_PEDGES,), jnp.int32),
        pltpu.VMEM_SHARED((_N_NODES, _D), jnp.float32),
    ),
)


def _dense_body(x_ref, p_ref, pd_ref, ws_ref, wn_ref, wc_ref, out_ref):
    nsum = p_ref[0] + p_ref[1]
    deg = jnp.sum(pd_ref[...], axis=1, keepdims=True)
    mean = nsum * (1.0 / jnp.maximum(deg, 1.0))
    e = lax.dot_general(x_ref[...], ws_ref[...], (((1,), (1,)), ((), ())),
                        preferred_element_type=jnp.float32)
    e += lax.dot_general(mean, wn_ref[...], (((1,), (1,)), ((), ())),
                         preferred_element_type=jnp.float32)
    e = jnp.maximum(e, 0.0)
    out_ref[...] = lax.dot_general(e, wc_ref[...], (((1,), (1,)), ((), ())),
                                   preferred_element_type=jnp.float32)


_BLK = 1000


def _dense_call(x, psum, pdeg, w_self, w_neigh, weight):
    grid = _N_NODES // _BLK
    return pl.pallas_call(
        _dense_body,
        grid=(grid,),
        in_specs=[
            pl.BlockSpec((_BLK, _D), lambda i: (i, 0)),
            pl.BlockSpec((_NC, _BLK, _D), lambda i: (0, i, 0)),
            pl.BlockSpec((_BLK, _NW), lambda i: (i, 0)),
            pl.BlockSpec((256, _D), lambda i: (0, 0)),
            pl.BlockSpec((256, _D), lambda i: (0, 0)),
            pl.BlockSpec((_D, 256), lambda i: (0, 0)),
        ],
        out_specs=pl.BlockSpec((_BLK, _D), lambda i: (i, 0)),
        out_shape=jax.ShapeDtypeStruct((_N_NODES, _D), jnp.float32),
    )(x, psum, pdeg, w_self, w_neigh, weight)


def _gather_body(nodes_ref, s_ref, out_ref, nall, grows, h0, h1, h2, h3):
    c = lax.axis_index("c")
    s = lax.axis_index("s")
    wid = s * _NC + c
    base = wid * _PERW
    sems = [h0, h1, h2, h3]
    pltpu.sync_copy(nodes_ref.at[pl.ds(base, _PERW)], nall)
    for j in range(_NGCH):
        pltpu.async_copy(s_ref.at[nall.at[pl.ds(j * _GCH, _GCH)]],
                         grows.at[j], sems[j])
    for j in range(_NGCH):
        pltpu.make_async_copy(s_ref.at[pl.ds(0, _GCH)],
                              grows.at[j], sems[j]).wait()
        pltpu.sync_copy(grows.at[j],
                        out_ref.at[pl.ds(base + j * _GCH, _GCH)])


_gather_call = pl.kernel(
    _gather_body,
    out_type=jax.ShapeDtypeStruct((_BPAD, _D), jnp.float32),
    mesh=_mesh,
    compiler_params=pltpu.CompilerParams(needs_layout_passes=False),
    scratch_types=(
        pltpu.VMEM((_PERW,), jnp.int32),
        pltpu.VMEM((_NGCH, _GCH, _D), jnp.float32),
        pltpu.SemaphoreType.DMA,
        pltpu.SemaphoreType.DMA,
        pltpu.SemaphoreType.DMA,
        pltpu.SemaphoreType.DMA,
    ),
)


def kernel(nodes, x, edge_index, W_enc, weight):
    edge = edge_index.astype(jnp.int32)
    nodes32 = nodes.astype(jnp.int32)
    zx = jnp.zeros((_EXP, _D), jnp.float32)
    zdeg = jnp.zeros((_N_NODES,), jnp.float32)
    psum, pdeg = _scatter_call(edge[0], edge[1], x, zx, zdeg)
    w_self = W_enc[:, :_D]
    w_neigh = W_enc[:, _D:]
    weight_pad = jnp.pad(weight, ((0, _D - _NCLS), (0, 0)))
    scores = _dense_call(x, psum, pdeg.T, w_self, w_neigh, weight_pad)
    nodes_pad = jnp.concatenate(
        [nodes32, jnp.zeros((_BPAD - _B,), jnp.int32)])
    out_pad = _gather_call(nodes_pad, scores)
    return out_pad[:_B, :_NCLS]


# speedup vs baseline: 1.0143x; 1.0143x over previous
import jax
import jax.numpy as jnp
from jax import lax
from jax.experimental import pallas as pl
from jax.experimental.pallas import tpu as pltpu
from jax.experimental.pallas import tpu_sc as plsc

_N_NODES = 10000
_N_EDGES = 320000
_D = 128
_NCLS = 64
_B = 10000

_NC = 2
_NS = 16
_NW = _NC * _NS

_PEDGES = _N_EDGES // _NW
_CHUNK = 40
_NCH = _PEDGES // _CHUNK
_NBUF = 6

_EXP = 624
_EXP_TAIL = _N_NODES - _NS * _EXP

_BPAD = 10240
_PERW = _BPAD // _NW
_GCH = 80
_NGCH = _PERW // _GCH

_mesh = plsc.VectorSubcoreMesh(
    core_axis_name="c", subcore_axis_name="s", num_cores=_NC, num_subcores=_NS)


def _scatter_body(esrc_ref, edst_ref, x_ref, zx_ref, zdeg_ref,
                  psum_ref, pdeg_ref, sall, dall, acc):
    c = lax.axis_index("c")
    s = lax.axis_index("s")
    wid = s * _NC + c

    row0 = s * _EXP
    pltpu.sync_copy(zx_ref, acc.at[pl.ds(row0, _EXP)])

    @pl.when(s == _NS - 1)
    def _init_tail():
        pltpu.sync_copy(zx_ref.at[pl.ds(0, _EXP_TAIL)],
                        acc.at[pl.ds(_NS * _EXP, _EXP_TAIL)])

    e0 = wid * _PEDGES
    pltpu.sync_copy(esrc_ref.at[pl.ds(e0, _PEDGES)], sall)
    pltpu.sync_copy(edst_ref.at[pl.ds(e0, _PEDGES)], dall)
    plsc.subcore_barrier()

    def _main(rows, *sems):
        gsem = list(sems[:_NBUF])
        ssem = list(sems[_NBUF:])

        def _issue(j, b):
            pltpu.async_copy(x_ref.at[sall.at[pl.ds(j * _CHUNK, _CHUNK)]],
                             rows.at[b], gsem[b])

        def _consume(j, b):
            pltpu.make_async_copy(x_ref.at[pl.ds(0, _CHUNK)],
                                  rows.at[b], gsem[b]).wait()
            pltpu.async_copy(rows.at[b],
                             acc.at[dall.at[pl.ds(j * _CHUNK, _CHUNK)]],
                             ssem[b], add=True)

        def _drain_scatter(b):
            pltpu.make_async_copy(rows.at[b], acc.at[pl.ds(0, _CHUNK)],
                                  ssem[b]).wait()

        for b in range(_NBUF):
            _issue(b, b)

        def group_step(g, carry):
            for b in range(_NBUF):
                j = g * _NBUF + b
                _consume(j, b)
                nxt = j + _NBUF

                @pl.when(nxt < _NCH)
                def _reissue():
                    _drain_scatter(b)
                    _issue(nxt, b)
            return carry
        lax.fori_loop(0, _NCH // _NBUF, group_step, 0)
        for t, j in enumerate(range((_NCH // _NBUF) * _NBUF, _NCH)):
            _consume(j, t)
        for b in range(_NBUF):
            _drain_scatter(b)

    pl.run_scoped(
        _main,
        pltpu.VMEM((_NBUF, _CHUNK, _D), jnp.float32),
        *([pltpu.SemaphoreType.DMA] * (2 * _NBUF)),
    )

    def _deg(degloc):
        pltpu.sync_copy(zdeg_ref, degloc)
        one16 = jnp.ones((16,), jnp.float32)

        def step(i, carry):
            dvec = dall[pl.ds(i * 16, 16)]
            plsc.addupdate_scatter(degloc, [dvec], one16)
            return carry
        lax.fori_loop(0, _PEDGES // 16, step, 0)
        pltpu.sync_copy(degloc, pdeg_ref.at[wid])

    pl.run_scoped(_deg, pltpu.VMEM((_N_NODES,), jnp.float32))

    plsc.subcore_barrier()
    pltpu.sync_copy(acc.at[pl.ds(row0, _EXP)],
                    psum_ref.at[c, pl.ds(row0, _EXP)])

    @pl.when(s == _NS - 1)
    def _exp_tail():
        pltpu.sync_copy(acc.at[pl.ds(_NS * _EXP, _EXP_TAIL)],
                        psum_ref.at[c, pl.ds(_NS * _EXP, _EXP_TAIL)])


_scatter_call = pl.kernel(
    _scatter_body,
    out_type=(
        jax.ShapeDtypeStruct((_NC, _N_NODES, _D), jnp.float32),
        jax.ShapeDtypeStruct((_NW, _N_NODES), jnp.float32),
    ),
    mesh=_mesh,
    compiler_params=pltpu.CompilerParams(needs_layout_passes=False),
    scratch_types=(
        pltpu.VMEM((_PEDGES,), jnp.int32),
        pltpu.VMEM((_PEDGES,), jnp.int32),
        pltpu.VMEM_SHARED((_N_NODES, _D), jnp.float32),
    ),
)


def _dense_body(x_ref, p_ref, pd_ref, ws_ref, wn_ref, wc_ref, out_ref):
    nsum = p_ref[0] + p_ref[1]
    deg = jnp.sum(pd_ref[...], axis=1, keepdims=True)
    mean = nsum * (1.0 / jnp.maximum(deg, 1.0))
    e = lax.dot_general(x_ref[...], ws_ref[...], (((1,), (1,)), ((), ())),
                        preferred_element_type=jnp.float32)
    e += lax.dot_general(mean, wn_ref[...], (((1,), (1,)), ((), ())),
                         preferred_element_type=jnp.float32)
    e = jnp.maximum(e, 0.0)
    out_ref[...] = lax.dot_general(e, wc_ref[...], (((1,), (1,)), ((), ())),
                                   preferred_element_type=jnp.float32)


_BLK = 1000


def _dense_call(x, psum, pdeg, w_self, w_neigh, weight):
    grid = _N_NODES // _BLK
    return pl.pallas_call(
        _dense_body,
        grid=(grid,),
        in_specs=[
            pl.BlockSpec((_BLK, _D), lambda i: (i, 0)),
            pl.BlockSpec((_NC, _BLK, _D), lambda i: (0, i, 0)),
            pl.BlockSpec((_BLK, _NW), lambda i: (i, 0)),
            pl.BlockSpec((256, _D), lambda i: (0, 0)),
            pl.BlockSpec((256, _D), lambda i: (0, 0)),
            pl.BlockSpec((_D, 256), lambda i: (0, 0)),
        ],
        out_specs=pl.BlockSpec((_BLK, _D), lambda i: (i, 0)),
        out_shape=jax.ShapeDtypeStruct((_N_NODES, _D), jnp.float32),
    )(x, psum, pdeg, w_self, w_neigh, weight)


def _gather_body(nodes_ref, s_ref, out_ref, nall, grows, h0, h1, h2, h3):
    c = lax.axis_index("c")
    s = lax.axis_index("s")
    wid = s * _NC + c
    base = wid * _PERW
    sems = [h0, h1, h2, h3]
    pltpu.sync_copy(nodes_ref.at[pl.ds(base, _PERW)], nall)
    for j in range(_NGCH):
        pltpu.async_copy(s_ref.at[nall.at[pl.ds(j * _GCH, _GCH)]],
                         grows.at[j], sems[j])
    for j in range(_NGCH):
        pltpu.make_async_copy(s_ref.at[pl.ds(0, _GCH)],
                              grows.at[j], sems[j]).wait()
        pltpu.sync_copy(grows.at[j],
                        out_ref.at[pl.ds(base + j * _GCH, _GCH)])


_gather_call = pl.kernel(
    _gather_body,
    out_type=jax.ShapeDtypeStruct((_BPAD, _D), jnp.float32),
    mesh=_mesh,
    compiler_params=pltpu.CompilerParams(needs_layout_passes=False),
    scratch_types=(
        pltpu.VMEM((_PERW,), jnp.int32),
        pltpu.VMEM((_NGCH, _GCH, _D), jnp.float32),
        pltpu.SemaphoreType.DMA,
        pltpu.SemaphoreType.DMA,
        pltpu.SemaphoreType.DMA,
        pltpu.SemaphoreType.DMA,
    ),
)


def kernel(nodes, x, edge_index, W_enc, weight):
    edge = edge_index.astype(jnp.int32)
    nodes32 = nodes.astype(jnp.int32)
    zx = jnp.zeros((_EXP, _D), jnp.float32)
    zdeg = jnp.zeros((_N_NODES,), jnp.float32)
    psum, pdeg = _scatter_call(edge[0], edge[1], x, zx, zdeg)
    w_self = W_enc[:, :_D]
    w_neigh = W_enc[:, _D:]
    weight_pad = jnp.pad(weight, ((0, _D - _NCLS), (0, 0)))
    scores = _dense_call(x, psum, pdeg.T, w_self, w_neigh, weight_pad)
    nodes_pad = jnp.concatenate(
        [nodes32, jnp.zeros((_BPAD - _B,), jnp.int32)])
    out_pad = _gather_call(nodes_pad, scores)
    return out_pad[:_B, :_NCLS]
